# baseline (speedup 1.0000x reference)
import functools

import jax
import jax.numpy as jnp
from jax import lax
from jax.experimental import pallas as pl
from jax.experimental.pallas import tpu as pltpu
from jax.experimental.pallas import tpu_sc as plsc

B, C, H, W, K = 32, 4, 128, 128, 128
HW = H * W
L = 16
G = 16


def _rsqrt_plus_eps_inv(s):
    xi = plsc.bitcast(s, jnp.int32)
    yi = jnp.int32(0x5F3759DF) - lax.shift_right_logical(xi, 1)
    y = plsc.bitcast(yi, jnp.float32)
    for _ in range(3):
        y = y * (1.5 - 0.5 * s * y * y)
    sqrt_s = s * y
    return 1.0 / (sqrt_s + 1e-10)


def _sc_distances(feat, tp, tq):
    info = plsc.get_sparse_core_info()
    nc = info.num_cores
    mesh = plsc.VectorSubcoreMesh(core_axis_name="c", subcore_axis_name="s")

    @functools.partial(
        pl.kernel,
        mesh=mesh,
        out_type=(
            jax.ShapeDtypeStruct((B, K), jnp.float32),
            jax.ShapeDtypeStruct((B, K), jnp.float32),
        ),
        compiler_params=pltpu.CompilerParams(needs_layout_passes=False),
        scratch_types=[
            pltpu.VMEM((2, K), jnp.int32),
            pltpu.VMEM((2, K), jnp.int32),
            pltpu.VMEM((G, K), jnp.int32),
            pltpu.VMEM((G, K), jnp.float32),
            pltpu.VMEM((K,), jnp.float32),
            pltpu.VMEM((K,), jnp.float32),
            pltpu.SemaphoreType.DMA,
        ],
    )
    def body(feat_hbm, tp_hbm, tq_hbm, outp_hbm, outq_hbm,
             tp_v, tq_v, idx_v, res_v, d2_v, pt_v, sem):
        wid = lax.axis_index("s") * nc + lax.axis_index("c")
        pltpu.sync_copy(tp_hbm.at[pl.ds(2 * wid, 2)], tp_v)
        pltpu.sync_copy(tq_hbm.at[pl.ds(2 * wid, 2)], tq_v)

        base = wid * C * HW

        def build(j, _):
            sl = pl.ds(j * L, L)
            for si, src in enumerate((tp_v.at[0], tp_v.at[1],
                                      tq_v.at[0], tq_v.at[1])):
                p = src[sl] + base
                for c in range(C):
                    idx_v[si * C + c, sl] = p + c * HW
            return 0

        lax.fori_loop(0, K // L, build, 0, unroll=False)

        copies = [pltpu.async_copy(feat_hbm.at[idx_v.at[g]], res_v.at[g], sem)
                  for g in range(G)]
        for cp in copies:
            cp.wait()

        def norm(g0, sl):
            fs = [res_v[g0 + c, sl] for c in range(C)]
            s = fs[0] * fs[0] + fs[1] * fs[1] + fs[2] * fs[2] + fs[3] * fs[3]
            r = _rsqrt_plus_eps_inv(s)
            return [f * r for f in fs]

        def chunk(j, _):
            sl = pl.ds(j * L, L)
            n0 = norm(0, sl)
            n1 = norm(C, sl)
            d2 = jnp.zeros((L,), jnp.float32)
            for a, b in zip(n0, n1):
                d = a - b
                d2 = d2 + d * d
            d2_v[sl] = d2

            p0 = norm(2 * C, sl)
            p1 = norm(3 * C, sl)
            l1 = jnp.zeros((L,), jnp.float32)
            for a, b in zip(p0, p1):
                l1 = l1 + jnp.abs(a - b)
            pt_v[sl] = jnp.maximum(1.0 - l1, 0.0)
            return 0

        lax.fori_loop(0, K // L, chunk, 0, unroll=False)

        pltpu.sync_copy(d2_v, outp_hbm.at[wid])
        pltpu.sync_copy(pt_v, outq_hbm.at[wid])

    return body(feat, tp, tq)


def _finalize_body(d2_ref, pt_ref, mp_ref, mq_ref, o_ref):
    mpf = mp_ref[...].astype(jnp.float32)
    mqf = mq_ref[...].astype(jnp.float32)
    ps = jnp.sum(d2_ref[...] * mpf)
    pc = jnp.sum(mpf)
    qs = jnp.sum(pt_ref[...] * mqf)
    qc = jnp.sum(mqf)
    loss = ps / (pc + 1e-4) + qs / (qc + 1e-4)
    o_ref[...] = jnp.full((1, 1), loss, jnp.float32)


def kernel(output, tag_pull, tag_push, mask_pull, mask_push):
    feat = output.reshape(B * C * HW)
    tp = tag_pull.transpose(0, 2, 1).reshape(2 * B, K)
    tq = tag_push.transpose(0, 2, 1).reshape(2 * B, K)
    d2, pt = _sc_distances(feat, tp, tq)
    loss = pl.pallas_call(
        _finalize_body,
        out_shape=jax.ShapeDtypeStruct((1, 1), jnp.float32),
    )(d2, pt, mask_pull, mask_push)
    return loss[0, 0]
